# Initial kernel scaffold; baseline (speedup 1.0000x reference)
import jax
import jax.numpy as jnp
from jax.experimental import pallas as pl


def kernel(x, pre_bias, latent_bias, W_enc, W_dec):
    raise NotImplementedError("write your pallas kernel here")


# speedup vs baseline: 2.7199x; 2.7199x over previous
import functools

import jax
import jax.numpy as jnp
import numpy as np
from jax import lax
from jax.experimental import pallas as pl
from jax.experimental.pallas import tpu as pltpu
from jax.experimental.pallas import tpu_sc as plsc

_EPS = 1e-05
_D = 768
_F = 32768
_K = 32
_N = 2048

_NC = 2
_NS = 16
_NW = _NC * _NS
_L = 16
_ROWS_PER_W = _N // _NW

_NEG = np.float32(-3.0e38)

_CAP = 544
_CVR = _CAP // _L
_BLK = 256
_NBLK = _F // _BLK
_COMPACT_AT = 256


_RB = 512
_FB = 2048


def _encode_body(x_ref, pb_ref, lb_ref, w_ref, h_ref):
    r = pl.program_id(1)
    xb = x_ref[pl.ds(r * _RB, _RB), :]
    mean = jnp.mean(xb, axis=1, keepdims=True)
    var = jnp.mean(jnp.square(xb - mean), axis=1, keepdims=True)
    inv = lax.rsqrt(var + _EPS)
    xc = (xb - mean) * inv - pb_ref[:][None, :]
    h = lax.dot_general(xc, w_ref[:], (((1,), (1,)), ((), ())),
                        preferred_element_type=jnp.float32)
    h_ref[:, :] = h + lb_ref[:][None, :]


def _encode(x, pre_bias, latent_bias, w_enc):
    return pl.pallas_call(
        _encode_body,
        grid=(_F // _FB, _N // _RB),
        in_specs=[
            pl.BlockSpec((_N, _D), lambda f, r: (0, 0)),
            pl.BlockSpec((_D,), lambda f, r: (0,)),
            pl.BlockSpec((_FB,), lambda f, r: (f,)),
            pl.BlockSpec((_FB, _D), lambda f, r: (f, 0)),
        ],
        out_specs=pl.BlockSpec((_RB, _FB), lambda f, r: (r, f)),
        out_shape=jax.ShapeDtypeStruct((_N, _F), jnp.float32),
    )(x, pre_bias, latent_bias, w_enc)


def _scalar(v):
    if getattr(v, "shape", ()) == ():
        return v
    return jnp.max(v)


def _splat(s, dtype=jnp.float32):
    return lax.broadcast(s.astype(dtype) if s.dtype != dtype else s, (_L,))


def _vreg(ref, start):
    return ref[pl.ds(start, _L)]


def _extract_top32(cand_v, cand_i, topk_v, topk_i, gm_v):
    for g in range(3):
        mx = _splat(jnp.float32(_NEG))
        for t in range(_L):
            j = g * _L + t
            if j < _CVR:
                val = jnp.max(_vreg(cand_v, j * _L))
                onehot = lax.iota(jnp.int32, _L) == t
                mx = jnp.where(onehot, _splat(val), mx)
        gm_v[pl.ds(g * _L, _L)] = mx

    def ext_body(k, _):
        g0 = _vreg(gm_v, 0)
        g1 = _vreg(gm_v, _L)
        g2 = _vreg(gm_v, 2 * _L)
        m = jnp.max(jnp.maximum(jnp.maximum(g0, g1), g2))
        msp = _splat(m)
        big = _splat(jnp.int32(10 ** 9), jnp.int32)
        io = lax.iota(jnp.int32, _L)
        pos = jnp.minimum(
            jnp.minimum(jnp.where(g0 == msp, io, big),
                        jnp.where(g1 == msp, io + _L, big)),
            jnp.where(g2 == msp, io + 2 * _L, big))
        j = jnp.min(pos)
        base = j * _L
        vv = _vreg(cand_v, base)
        lane_pos = jnp.min(jnp.where(vv == msp, io, big))
        ii = _vreg(cand_i, base)
        idx_val = jnp.max(jnp.where(io == lane_pos, ii, jnp.int32(-1)))
        k_one = io == (k % _L)
        half = k // _L
        tv = _vreg(topk_v, half * _L)
        ti = _vreg(topk_i, half * _L)
        topk_v[pl.ds(half * _L, _L)] = jnp.where(k_one, msp, tv)
        topk_i[pl.ds(half * _L, _L)] = jnp.where(k_one, _splat(idx_val,
                                                               jnp.int32), ti)
        vv2 = jnp.where(io == lane_pos, _splat(jnp.float32(_NEG)), vv)
        cand_v[pl.ds(base, _L)] = vv2
        newmax = jnp.max(vv2)
        gsel = j // _L
        glane = j % _L
        for g in range(3):
            cur = _vreg(gm_v, g * _L)
            upd = jnp.where((io == glane) & (gsel == g), _splat(newmax), cur)
            gm_v[pl.ds(g * _L, _L)] = upd
        return 0

    lax.fori_loop(0, _K, ext_body, 0, unroll=False)


def _sc_body(hpre_hbm, wdt_hbm, x_hbm, pb_hbm, out_hbm,
             row_v, cand_v, cand_i, topk_v, topk_i, gm_v,
             gbuf_v, xrow_v, orow_v, pb_v, sem):
    wid = lax.axis_index("s") * _NC + lax.axis_index("c")
    base_row = wid * _ROWS_PER_W
    pltpu.sync_copy(pb_hbm, pb_v)

    def row_body(i, _):
        r = base_row + i
        pltpu.sync_copy(hpre_hbm.at[r], row_v)
        pltpu.sync_copy(x_hbm.at[r], xrow_v)

        for j in range(_CVR):
            cand_v[pl.ds(j * _L, _L)] = _splat(jnp.float32(_NEG))

        def blk_body(b, carry):
            thr, cnt = carry
            base = b * _BLK
            bm = _vreg(row_v, base)
            for j in range(1, _BLK // _L):
                bm = jnp.maximum(bm, _vreg(row_v, base + j * _L))
            hit = jnp.max(bm) > thr

            def rare(c):
                thr_, cnt_ = c
                tsp = _splat(thr_)
                cc = cnt_
                for j in range(_BLK // _L):
                    v = _vreg(row_v, base + j * _L)
                    msk = v > tsp
                    idx = lax.iota(jnp.int32, _L) + (base + j * _L)
                    plsc.store_compressed(cand_v.at[pl.ds(cc, _L)], v,
                                          mask=msk)
                    plsc.store_compressed(cand_i.at[pl.ds(cc, _L)], idx,
                                          mask=msk)
                    cc = cc + _scalar(plsc.all_reduce_population_count(msk))
                return thr_, cc

            thr, cnt = lax.cond(hit, rare, lambda c: c, (thr, cnt))

            def compact(c):
                thr_, cnt_ = c
                _extract_top32(cand_v, cand_i, topk_v, topk_i, gm_v)
                for j in range(_CVR):
                    cand_v[pl.ds(j * _L, _L)] = _splat(jnp.float32(_NEG))
                cand_v[pl.ds(0, _L)] = _vreg(topk_v, 0)
                cand_v[pl.ds(_L, _L)] = _vreg(topk_v, _L)
                cand_i[pl.ds(0, _L)] = _vreg(topk_i, 0)
                cand_i[pl.ds(_L, _L)] = _vreg(topk_i, _L)
                new_thr = jnp.min(jnp.minimum(_vreg(topk_v, 0),
                                              _vreg(topk_v, _L)))
                return new_thr, jnp.int32(_K)

            return lax.cond(cnt >= _COMPACT_AT, compact, lambda c: c,
                            (thr, cnt))

        lax.fori_loop(0, _NBLK, blk_body, (_NEG, jnp.int32(0)),
                      unroll=False)

        _extract_top32(cand_v, cand_i, topk_v, topk_i, gm_v)

        gat = pltpu.async_copy(wdt_hbm.at[topk_i], gbuf_v, sem)
        gat.wait()

        zero = _splat(jnp.float32(0.0))
        relu0 = jnp.maximum(_vreg(topk_v, 0), 0.0)
        relu1 = jnp.maximum(_vreg(topk_v, _L), 0.0)
        for c in range(3):
            c0 = c * 256

            def acc_body(k, acc):
                vhalf = jnp.where(k < _L, relu0, relu1)
                lane = lax.broadcast(k % _L, (_L,))
                vsp = vhalf.at[lane].get(mode="promise_in_bounds")
                return tuple(
                    acc[j] + vsp * gbuf_v[k, pl.ds(c0 + j * _L, _L)]
                    for j in range(16))

            acc = lax.fori_loop(0, _K, acc_body, (zero,) * 16, unroll=False)
            for j in range(16):
                orow_v[pl.ds(c0 + j * _L, _L)] = acc[j]

        s = zero
        for j in range(_D // _L):
            s = s + _vreg(xrow_v, j * _L)
        mean = jnp.sum(s) * jnp.float32(1.0 / _D)
        msp = _splat(mean)
        s2 = zero
        for j in range(_D // _L):
            dlt = _vreg(xrow_v, j * _L) - msp
            s2 = s2 + dlt * dlt
        var = jnp.sum(s2) * jnp.float32(1.0 / _D) + jnp.float32(_EPS)
        vv = _splat(var)
        bits = plsc.bitcast(vv, jnp.int32)
        y = plsc.bitcast(jnp.int32(0x5F3759DF) - (bits >> 1), jnp.float32)
        half = _splat(jnp.float32(0.5)) * vv
        for _it in range(3):
            y = y * (_splat(jnp.float32(1.5)) - half * y * y)
        std = vv * y
        for j in range(_D // _L):
            o = _vreg(orow_v, j * _L) + _vreg(pb_v, j * _L)
            orow_v[pl.ds(j * _L, _L)] = o * std + msp
        pltpu.sync_copy(orow_v, out_hbm.at[r])
        return 0

    lax.fori_loop(0, _ROWS_PER_W, row_body, 0, unroll=False)


def _sc_decode(h_pre, wdt, x, pre_bias):
    mesh = plsc.VectorSubcoreMesh(core_axis_name="c", subcore_axis_name="s",
                                  num_cores=_NC, num_subcores=_NS)
    f = pl.kernel(
        _sc_body,
        out_type=jax.ShapeDtypeStruct((_N, _D), jnp.float32),
        mesh=mesh,
        compiler_params=pltpu.CompilerParams(needs_layout_passes=False),
        scratch_types=[
            pltpu.VMEM((_F,), jnp.float32),
            pltpu.VMEM((_CAP,), jnp.float32),
            pltpu.VMEM((_CAP,), jnp.int32),
            pltpu.VMEM((_K,), jnp.float32),
            pltpu.VMEM((_K,), jnp.int32),
            pltpu.VMEM((48,), jnp.float32),
            pltpu.VMEM((_K, _D), jnp.float32),
            pltpu.VMEM((_D,), jnp.float32),
            pltpu.VMEM((_D,), jnp.float32),
            pltpu.VMEM((_D,), jnp.float32),
            pltpu.SemaphoreType.DMA,
        ],
    )
    return f(h_pre, wdt, x, pre_bias)


def kernel(x, pre_bias, latent_bias, W_enc, W_dec):
    h_pre = _encode(x, pre_bias, latent_bias, W_enc)
    wdt = W_dec.T
    return _sc_decode(h_pre, wdt, x, pre_bias)
